# Initial kernel scaffold; baseline (speedup 1.0000x reference)
import jax
import jax.numpy as jnp
from jax.experimental import pallas as pl


def kernel(x, table, W, b):
    raise NotImplementedError("write your pallas kernel here")


# speedup vs baseline: 2.0848x; 2.0848x over previous
import functools

import jax
import jax.numpy as jnp
from jax import lax
from jax.experimental import pallas as pl
from jax.experimental.pallas import tpu as pltpu
import jax.experimental.pallas.tpu_sc as plsc

_NC, _NS = 2, 16
_NW = _NC * _NS
_ROWS_PER_BLK = 8000
_CHUNK = 128
_FIRE = 8


def _score_body(tab_ref, w_ref, out_ref):
    prod = tab_ref[...] * w_ref[...]
    s = jnp.sum(prod, axis=1, keepdims=True)
    rows = (pl.program_id(0) * _ROWS_PER_BLK
            + lax.broadcasted_iota(jnp.int32, (_ROWS_PER_BLK, 1), 0))
    out_ref[0] = jnp.where(rows == 0, 0.0, s).reshape(1, _ROWS_PER_BLK)


def _score_table(table, w):
    v, d = table.shape
    nblk = v // _ROWS_PER_BLK
    assert nblk * _ROWS_PER_BLK == v
    out = pl.pallas_call(
        _score_body,
        grid=(nblk,),
        in_specs=[
            pl.BlockSpec((_ROWS_PER_BLK, d), lambda i: (i, 0)),
            pl.BlockSpec((1, d), lambda i: (0, 0)),
        ],
        out_specs=pl.BlockSpec((1, 1, _ROWS_PER_BLK), lambda i: (i, 0, 0)),
        out_shape=jax.ShapeDtypeStruct((nblk, 1, _ROWS_PER_BLK), jnp.float32),
    )(table, w)
    return out.reshape(v)


def _make_sc_argmax(batch, seq, rows_per_w):
    n_idx = rows_per_w * seq
    n_chunks = n_idx // _CHUNK
    assert n_chunks * _CHUNK == n_idx and rows_per_w % 16 == 0
    mesh = plsc.VectorSubcoreMesh(core_axis_name="c", subcore_axis_name="s")

    @functools.partial(
        pl.kernel,
        out_type=jax.ShapeDtypeStruct((batch,), jnp.int32),
        mesh=mesh,
        compiler_params=pltpu.CompilerParams(needs_layout_passes=False),
        scratch_types=[
            pltpu.VMEM((n_chunks, _CHUNK), jnp.int32),
            pltpu.VMEM((n_idx,), jnp.float32),
            pltpu.VMEM((rows_per_w,), jnp.int32),
            pltpu.SemaphoreType.DMA,
        ],
    )
    def sc_argmax(xr_hbm, s_hbm, out_hbm, idx_v, vals_v, amax_v, sem):
        wid = lax.axis_index("s") * _NC + lax.axis_index("c")
        pltpu.sync_copy(xr_hbm.at[wid], idx_v)

        @pl.loop(0, n_chunks, step=_FIRE)
        def _gather(j0):
            copies = []
            for k in range(_FIRE):
                j = j0 + k
                copies.append(pltpu.async_copy(
                    s_hbm.at[idx_v.at[j]],
                    vals_v.at[pl.ds(j * _CHUNK, _CHUNK)], sem))
            for c in copies:
                c.wait()

        iota16 = lax.iota(jnp.int32, 16)

        @pl.loop(0, rows_per_w // 16)
        def _rows(g):
            off = (g * 16 + iota16) * seq
            maxv = plsc.load_gather(vals_v, [off])
            amax = jnp.zeros((16,), jnp.int32)
            for l in range(1, seq):
                val = plsc.load_gather(vals_v, [off + l])
                upd = val > maxv
                maxv = jnp.where(upd, val, maxv)
                amax = jnp.where(upd, jnp.full((16,), l, jnp.int32), amax)
            amax_v[pl.ds(g * 16, 16)] = amax

        pltpu.sync_copy(amax_v, out_hbm.at[pl.ds(wid * rows_per_w, rows_per_w)])

    return sc_argmax


def kernel(x, table, W, b):
    batch, seq = x.shape
    s_flat = _score_table(table, W)
    rows_per_w = batch // _NW
    n_chunks = (rows_per_w * seq) // _CHUNK
    x_r = x.astype(jnp.int32).reshape(_NW, n_chunks, _CHUNK)
    return _make_sc_argmax(batch, seq, rows_per_w)(x_r, s_flat)


# speedup vs baseline: 10.1812x; 4.8835x over previous
import functools

import jax
import jax.numpy as jnp
from jax import lax
from jax.experimental import pallas as pl
from jax.experimental.pallas import tpu as pltpu
import jax.experimental.pallas.tpu_sc as plsc

_NC, _NS = 2, 16
_NW = _NC * _NS
_CHUNK = 128
_FIRE = 8
_LANE_CHUNK = 131072
_K_TILE = 8
_VPAD = 1 << 20


def _score_body(tab_ref, w_ref, out_ref, acc_ref):
    j = pl.program_id(1)

    @pl.when(j == 0)
    def _():
        acc_ref[...] = tab_ref[...] * w_ref[...]

    @pl.when(j > 0)
    def _():
        acc_ref[...] += tab_ref[...] * w_ref[...]

    @pl.when(j == pl.num_programs(1) - 1)
    def _():
        s = jnp.sum(acc_ref[...], axis=0, keepdims=True)
        vv = (pl.program_id(0) * _LANE_CHUNK
              + lax.broadcasted_iota(jnp.int32, (1, _LANE_CHUNK), 1))
        out_ref[...] = jnp.where(vv == 0, 0.0, s)


def _score_table(tab_t, w_col):
    d, v = tab_t.shape
    out = pl.pallas_call(
        _score_body,
        grid=(_VPAD // _LANE_CHUNK, d // _K_TILE),
        in_specs=[
            pl.BlockSpec((_K_TILE, _LANE_CHUNK), lambda i, j: (j, i)),
            pl.BlockSpec((_K_TILE, 1), lambda i, j: (j, 0)),
        ],
        out_specs=pl.BlockSpec((1, _LANE_CHUNK), lambda i, j: (0, i)),
        out_shape=jax.ShapeDtypeStruct((1, _VPAD), jnp.float32),
        scratch_shapes=[pltpu.VMEM((_K_TILE, _LANE_CHUNK), jnp.float32)],
    )(tab_t, w_col)
    return out.reshape(_VPAD)


def _make_sc_argmax(batch, seq, rows_per_w):
    n_idx = rows_per_w * seq
    n_chunks = n_idx // _CHUNK
    assert n_chunks * _CHUNK == n_idx and rows_per_w % 16 == 0
    mesh = plsc.VectorSubcoreMesh(core_axis_name="c", subcore_axis_name="s")

    @functools.partial(
        pl.kernel,
        out_type=jax.ShapeDtypeStruct((batch,), jnp.int32),
        mesh=mesh,
        compiler_params=pltpu.CompilerParams(needs_layout_passes=False),
        scratch_types=[
            pltpu.VMEM((n_chunks, _CHUNK), jnp.int32),
            pltpu.VMEM((n_idx,), jnp.float32),
            pltpu.VMEM((rows_per_w,), jnp.int32),
            pltpu.SemaphoreType.DMA,
        ],
    )
    def sc_argmax(xr_hbm, s_hbm, out_hbm, idx_v, vals_v, amax_v, sem):
        wid = lax.axis_index("s") * _NC + lax.axis_index("c")
        pltpu.sync_copy(xr_hbm.at[wid], idx_v)

        @pl.loop(0, n_chunks, step=_FIRE)
        def _gather(j0):
            copies = []
            for k in range(_FIRE):
                j = j0 + k
                copies.append(pltpu.async_copy(
                    s_hbm.at[idx_v.at[j]],
                    vals_v.at[pl.ds(j * _CHUNK, _CHUNK)], sem))
            for c in copies:
                c.wait()

        iota16 = lax.iota(jnp.int32, 16)

        @pl.loop(0, rows_per_w // 16)
        def _rows(g):
            off = (g * 16 + iota16) * seq
            maxv = plsc.load_gather(vals_v, [off])
            amax = jnp.zeros((16,), jnp.int32)
            for l in range(1, seq):
                val = plsc.load_gather(vals_v, [off + l])
                upd = val > maxv
                maxv = jnp.where(upd, val, maxv)
                amax = jnp.where(upd, jnp.full((16,), l, jnp.int32), amax)
            amax_v[pl.ds(g * 16, 16)] = amax

        pltpu.sync_copy(amax_v, out_hbm.at[pl.ds(wid * rows_per_w, rows_per_w)])

    return sc_argmax


def kernel(x, table, W, b):
    batch, seq = x.shape
    s_flat = _score_table(table.T, W.reshape(-1, 1))
    rows_per_w = batch // _NW
    n_chunks = (rows_per_w * seq) // _CHUNK
    x_r = x.astype(jnp.int32).reshape(_NW, n_chunks, _CHUNK)
    return _make_sc_argmax(batch, seq, rows_per_w)(x_r, s_flat)


# speedup vs baseline: 10.4141x; 1.0229x over previous
import functools

import jax
import jax.numpy as jnp
from jax import lax
from jax.experimental import pallas as pl
from jax.experimental.pallas import tpu as pltpu
import jax.experimental.pallas.tpu_sc as plsc

_NC, _NS = 2, 16
_NW = _NC * _NS
_CHUNK = 128
_DEPTH = 16
_LANE_CHUNK = 131072
_K_TILE = 8
_VPAD = 1 << 20


def _score_body(tab_ref, w_ref, out_ref, acc_ref):
    j = pl.program_id(1)

    @pl.when(j == 0)
    def _():
        acc_ref[...] = tab_ref[...] * w_ref[...]

    @pl.when(j > 0)
    def _():
        acc_ref[...] += tab_ref[...] * w_ref[...]

    @pl.when(j == pl.num_programs(1) - 1)
    def _():
        s = jnp.sum(acc_ref[...], axis=0, keepdims=True)
        vv = (pl.program_id(0) * _LANE_CHUNK
              + lax.broadcasted_iota(jnp.int32, (1, _LANE_CHUNK), 1))
        out_ref[...] = jnp.where(vv == 0, 0.0, s)


def _score_table(tab_t, w_col):
    d, v = tab_t.shape
    out = pl.pallas_call(
        _score_body,
        grid=(_VPAD // _LANE_CHUNK, d // _K_TILE),
        in_specs=[
            pl.BlockSpec((_K_TILE, _LANE_CHUNK), lambda i, j: (j, i)),
            pl.BlockSpec((_K_TILE, 1), lambda i, j: (j, 0)),
        ],
        out_specs=pl.BlockSpec((1, _LANE_CHUNK), lambda i, j: (0, i)),
        out_shape=jax.ShapeDtypeStruct((1, _VPAD), jnp.float32),
        scratch_shapes=[pltpu.VMEM((_K_TILE, _LANE_CHUNK), jnp.float32)],
    )(tab_t, w_col)
    return out.reshape(_VPAD)


def _make_sc_argmax(batch, seq, rows_per_w):
    n_idx = rows_per_w * seq
    n_chunks = n_idx // _CHUNK
    assert n_chunks * _CHUNK == n_idx and rows_per_w % 16 == 0
    mesh = plsc.VectorSubcoreMesh(core_axis_name="c", subcore_axis_name="s")

    @functools.partial(
        pl.kernel,
        out_type=jax.ShapeDtypeStruct((batch,), jnp.int32),
        mesh=mesh,
        compiler_params=pltpu.CompilerParams(needs_layout_passes=False),
        scratch_types=[
            pltpu.VMEM((n_chunks, _CHUNK), jnp.int32),
            pltpu.VMEM((n_chunks, _CHUNK), jnp.float32),
            pltpu.VMEM((rows_per_w,), jnp.int32),
            pltpu.SemaphoreType.DMA,
        ],
    )
    def sc_argmax(xr_hbm, s_hbm, out_hbm, idx_v, vals_v, amax_v, sem):
        wid = lax.axis_index("s") * _NC + lax.axis_index("c")
        pltpu.sync_copy(xr_hbm.at[wid], idx_v)
        for k in range(_DEPTH):
            pltpu.async_copy(s_hbm.at[idx_v.at[k]], vals_v.at[k], sem)

        @pl.loop(0, n_chunks - _DEPTH)
        def _pipe(j):
            pltpu.make_async_copy(s_hbm.at[idx_v.at[j]], vals_v.at[j], sem).wait()
            jn = j + _DEPTH
            pltpu.async_copy(s_hbm.at[idx_v.at[jn]], vals_v.at[jn], sem)

        for k in range(_DEPTH):
            pltpu.make_async_copy(s_hbm.at[idx_v.at[k]], vals_v.at[k], sem).wait()

        iota16 = lax.iota(jnp.int32, 16)

        @pl.loop(0, rows_per_w // 16)
        def _rows(g):
            off = (g * 16 + iota16) * seq
            maxv = plsc.load_gather(vals_v, [off >> 7, off & 127])
            amax = jnp.zeros((16,), jnp.int32)
            for l in range(1, seq):
                o = off + l
                val = plsc.load_gather(vals_v, [o >> 7, o & 127])
                upd = val > maxv
                maxv = jnp.where(upd, val, maxv)
                amax = jnp.where(upd, jnp.full((16,), l, jnp.int32), amax)
            amax_v[pl.ds(g * 16, 16)] = amax

        pltpu.sync_copy(amax_v, out_hbm.at[pl.ds(wid * rows_per_w, rows_per_w)])

    return sc_argmax


def kernel(x, table, W, b):
    batch, seq = x.shape
    s_flat = _score_table(table.T, W.reshape(-1, 1))
    rows_per_w = batch // _NW
    n_chunks = (rows_per_w * seq) // _CHUNK
    x_r = x.astype(jnp.int32).reshape(_NW, n_chunks, _CHUNK)
    return _make_sc_argmax(batch, seq, rows_per_w)(x_r, s_flat)


# speedup vs baseline: 11.3873x; 1.0934x over previous
import functools

import jax
import jax.numpy as jnp
from jax import lax
from jax.experimental import pallas as pl
from jax.experimental.pallas import tpu as pltpu
import jax.experimental.pallas.tpu_sc as plsc

_NC, _NS = 2, 16
_NW = _NC * _NS
_CHUNK = 128
_DEPTH = 16
_LANE_CHUNK = 262144
_K_TILE = 8
_VPAD = 1 << 20


def _score_body(tab_ref, w_ref, out_ref, acc_ref):
    j = pl.program_id(1)

    @pl.when(j == 0)
    def _():
        acc_ref[...] = tab_ref[...] * w_ref[...]

    @pl.when(j > 0)
    def _():
        acc_ref[...] += tab_ref[...] * w_ref[...]

    @pl.when(j == pl.num_programs(1) - 1)
    def _():
        s = jnp.sum(acc_ref[...], axis=0, keepdims=True)
        vv = (pl.program_id(0) * _LANE_CHUNK
              + lax.broadcasted_iota(jnp.int32, (1, _LANE_CHUNK), 1))
        out_ref[...] = jnp.where(vv == 0, 0.0, s)


def _score_table(tab_t, w_col):
    d, v = tab_t.shape
    out = pl.pallas_call(
        _score_body,
        grid=(_VPAD // _LANE_CHUNK, d // _K_TILE),
        in_specs=[
            pl.BlockSpec((_K_TILE, _LANE_CHUNK), lambda i, j: (j, i)),
            pl.BlockSpec((_K_TILE, 1), lambda i, j: (j, 0)),
        ],
        out_specs=pl.BlockSpec((1, _LANE_CHUNK), lambda i, j: (0, i)),
        out_shape=jax.ShapeDtypeStruct((1, _VPAD), jnp.float32),
        scratch_shapes=[pltpu.VMEM((_K_TILE, _LANE_CHUNK), jnp.float32)],
    )(tab_t, w_col)
    return out.reshape(_VPAD)


def _make_sc_argmax(batch, seq, rows_per_w):
    n_idx = rows_per_w * seq
    n_chunks = n_idx // _CHUNK
    assert n_chunks * _CHUNK == n_idx and rows_per_w % 16 == 0
    mesh = plsc.VectorSubcoreMesh(core_axis_name="c", subcore_axis_name="s")

    @functools.partial(
        pl.kernel,
        out_type=jax.ShapeDtypeStruct((batch,), jnp.int32),
        mesh=mesh,
        compiler_params=pltpu.CompilerParams(needs_layout_passes=False),
        scratch_types=[
            pltpu.VMEM((n_chunks, _CHUNK), jnp.int32),
            pltpu.VMEM((n_chunks, _CHUNK), jnp.float32),
            pltpu.VMEM((rows_per_w,), jnp.int32),
            pltpu.SemaphoreType.DMA,
        ],
    )
    def sc_argmax(xr_hbm, s_hbm, out_hbm, idx_v, vals_v, amax_v, sem):
        wid = lax.axis_index("s") * _NC + lax.axis_index("c")
        pltpu.sync_copy(xr_hbm.at[wid], idx_v)
        for k in range(_DEPTH):
            pltpu.async_copy(s_hbm.at[idx_v.at[k]], vals_v.at[k], sem)

        @pl.loop(0, n_chunks - _DEPTH)
        def _pipe(j):
            pltpu.make_async_copy(s_hbm.at[idx_v.at[j]], vals_v.at[j], sem).wait()
            jn = j + _DEPTH
            pltpu.async_copy(s_hbm.at[idx_v.at[jn]], vals_v.at[jn], sem)

        for k in range(_DEPTH):
            pltpu.make_async_copy(s_hbm.at[idx_v.at[k]], vals_v.at[k], sem).wait()

        iota16 = lax.iota(jnp.int32, 16)

        @pl.loop(0, rows_per_w // 16)
        def _rows(g):
            off = (g * 16 + iota16) * seq
            maxv = plsc.load_gather(vals_v, [off >> 7, off & 127])
            amax = jnp.zeros((16,), jnp.int32)
            for l in range(1, seq):
                o = off + l
                val = plsc.load_gather(vals_v, [o >> 7, o & 127])
                upd = val > maxv
                maxv = jnp.where(upd, val, maxv)
                amax = jnp.where(upd, jnp.full((16,), l, jnp.int32), amax)
            amax_v[pl.ds(g * 16, 16)] = amax

        pltpu.sync_copy(amax_v, out_hbm.at[pl.ds(wid * rows_per_w, rows_per_w)])

    return sc_argmax


def kernel(x, table, W, b):
    batch, seq = x.shape
    s_flat = _score_table(table.T, W.reshape(-1, 1))
    rows_per_w = batch // _NW
    n_chunks = (rows_per_w * seq) // _CHUNK
    x_r = x.astype(jnp.int32).reshape(_NW, n_chunks, _CHUNK)
    return _make_sc_argmax(batch, seq, rows_per_w)(x_r, s_flat)


# speedup vs baseline: 13.5130x; 1.1867x over previous
import functools

import jax
import jax.numpy as jnp
from jax import lax
from jax.experimental import pallas as pl
from jax.experimental.pallas import tpu as pltpu
import jax.experimental.pallas.tpu_sc as plsc

_NC, _NS = 2, 16
_NW = _NC * _NS
_CHUNK = 128
_DEPTH = 16
_LANE_CHUNK = 262144
_K_TILE = 8
_VPAD = 1 << 20


def _score_body(tab_ref, w_ref, out_ref, acc_ref):
    j = pl.program_id(1)

    @pl.when(j == 0)
    def _():
        acc_ref[...] = tab_ref[...] * w_ref[...]

    @pl.when(j > 0)
    def _():
        acc_ref[...] += tab_ref[...] * w_ref[...]

    @pl.when(j == pl.num_programs(1) - 1)
    def _():
        s = jnp.sum(acc_ref[...], axis=0, keepdims=True)
        vv = (pl.program_id(0) * _LANE_CHUNK
              + lax.broadcasted_iota(jnp.int32, (1, _LANE_CHUNK), 1))
        out_ref[...] = jnp.where(vv == 0, 0.0, s)


def _score_table(tab_t, w_col):
    d, v = tab_t.shape
    out = pl.pallas_call(
        _score_body,
        grid=(_VPAD // _LANE_CHUNK, d // _K_TILE),
        in_specs=[
            pl.BlockSpec((_K_TILE, _LANE_CHUNK), lambda i, j: (j, i)),
            pl.BlockSpec((_K_TILE, 1), lambda i, j: (j, 0)),
        ],
        out_specs=pl.BlockSpec((1, _LANE_CHUNK), lambda i, j: (0, i)),
        out_shape=jax.ShapeDtypeStruct((1, _VPAD), jnp.float32),
        scratch_shapes=[pltpu.VMEM((_K_TILE, _LANE_CHUNK), jnp.float32)],
    )(tab_t, w_col)
    return out.reshape(_VPAD)


def _make_sc_argmax(batch, seq, rows_per_w):
    n_idx = rows_per_w * seq
    n_chunks = n_idx // _CHUNK
    assert n_chunks * _CHUNK == n_idx and rows_per_w % 16 == 0
    mesh = plsc.VectorSubcoreMesh(core_axis_name="c", subcore_axis_name="s")

    n_sub = _NS

    @functools.partial(
        pl.kernel,
        out_type=jax.ShapeDtypeStruct((batch,), jnp.int32),
        mesh=mesh,
        compiler_params=pltpu.CompilerParams(needs_layout_passes=False),
        scratch_types=[
            pltpu.VMEM((n_chunks, _CHUNK), jnp.int32),
            pltpu.VMEM((n_chunks, _CHUNK), jnp.float32),
            pltpu.VMEM((rows_per_w,), jnp.int32),
            pltpu.VMEM_SHARED((_VPAD,), jnp.float32),
            pltpu.SemaphoreType.DMA,
        ],
    )
    def sc_argmax(xr_hbm, s_hbm, out_hbm, idx_v, vals_v, amax_v, s_sh, sem):
        wid = lax.axis_index("s") * _NC + lax.axis_index("c")
        pltpu.sync_copy(xr_hbm.at[wid], idx_v)
        sl = _VPAD // n_sub
        sid = lax.axis_index("s")
        pltpu.sync_copy(s_hbm.at[pl.ds(sid * sl, sl)],
                        s_sh.at[pl.ds(sid * sl, sl)])
        plsc.subcore_barrier()
        for k in range(_DEPTH):
            pltpu.async_copy(s_sh.at[idx_v.at[k]], vals_v.at[k], sem)

        @pl.loop(0, n_chunks - _DEPTH)
        def _pipe(j):
            pltpu.make_async_copy(s_sh.at[idx_v.at[j]], vals_v.at[j], sem).wait()
            jn = j + _DEPTH
            pltpu.async_copy(s_sh.at[idx_v.at[jn]], vals_v.at[jn], sem)

        for k in range(_DEPTH):
            pltpu.make_async_copy(s_sh.at[idx_v.at[k]], vals_v.at[k], sem).wait()

        iota16 = lax.iota(jnp.int32, 16)

        @pl.loop(0, rows_per_w // 16)
        def _rows(g):
            off = (g * 16 + iota16) * seq
            maxv = plsc.load_gather(vals_v, [off >> 7, off & 127])
            amax = jnp.zeros((16,), jnp.int32)
            for l in range(1, seq):
                o = off + l
                val = plsc.load_gather(vals_v, [o >> 7, o & 127])
                upd = val > maxv
                maxv = jnp.where(upd, val, maxv)
                amax = jnp.where(upd, jnp.full((16,), l, jnp.int32), amax)
            amax_v[pl.ds(g * 16, 16)] = amax

        pltpu.sync_copy(amax_v, out_hbm.at[pl.ds(wid * rows_per_w, rows_per_w)])

    return sc_argmax


def kernel(x, table, W, b):
    batch, seq = x.shape
    s_flat = _score_table(table.T, W.reshape(-1, 1))
    rows_per_w = batch // _NW
    n_chunks = (rows_per_w * seq) // _CHUNK
    x_r = x.astype(jnp.int32).reshape(_NW, n_chunks, _CHUNK)
    return _make_sc_argmax(batch, seq, rows_per_w)(x_r, s_flat)


# speedup vs baseline: 14.7635x; 1.0925x over previous
import functools

import jax
import jax.numpy as jnp
from jax import lax
from jax.experimental import pallas as pl
from jax.experimental.pallas import tpu as pltpu
import jax.experimental.pallas.tpu_sc as plsc

_NC, _NS = 2, 16
_NW = _NC * _NS
_CHUNK = 128
_DEPTH = 16
_LANE_CHUNK = 65536
_VPAD = 1 << 20


def _score_body(tab_ref, w_ref, out_ref):
    s = jnp.sum(tab_ref[...] * w_ref[...], axis=0, keepdims=True)
    vv = (pl.program_id(0) * _LANE_CHUNK
          + lax.broadcasted_iota(jnp.int32, (1, _LANE_CHUNK), 1))
    out_ref[...] = jnp.where(vv == 0, 0.0, s)


def _score_table(tab_t, w_col):
    d, v = tab_t.shape
    out = pl.pallas_call(
        _score_body,
        grid=(_VPAD // _LANE_CHUNK,),
        in_specs=[
            pl.BlockSpec((d, _LANE_CHUNK), lambda i: (0, i)),
            pl.BlockSpec((d, 1), lambda i: (0, 0)),
        ],
        out_specs=pl.BlockSpec((1, _LANE_CHUNK), lambda i: (0, i)),
        out_shape=jax.ShapeDtypeStruct((1, _VPAD), jnp.float32),
    )(tab_t, w_col)
    return out.reshape(_VPAD)


def _make_sc_argmax(batch, seq, rows_per_w):
    n_idx = rows_per_w * seq
    n_chunks = n_idx // _CHUNK
    assert n_chunks * _CHUNK == n_idx and rows_per_w % 16 == 0
    mesh = plsc.VectorSubcoreMesh(core_axis_name="c", subcore_axis_name="s")

    n_sub = _NS

    @functools.partial(
        pl.kernel,
        out_type=jax.ShapeDtypeStruct((batch,), jnp.int32),
        mesh=mesh,
        compiler_params=pltpu.CompilerParams(needs_layout_passes=False),
        scratch_types=[
            pltpu.VMEM((n_chunks, _CHUNK), jnp.int32),
            pltpu.VMEM((n_chunks, _CHUNK), jnp.float32),
            pltpu.VMEM((rows_per_w,), jnp.int32),
            pltpu.VMEM_SHARED((_VPAD,), jnp.float32),
            pltpu.SemaphoreType.DMA,
        ],
    )
    def sc_argmax(xr_hbm, s_hbm, out_hbm, idx_v, vals_v, amax_v, s_sh, sem):
        wid = lax.axis_index("s") * _NC + lax.axis_index("c")
        pltpu.sync_copy(xr_hbm.at[wid], idx_v)
        sl = _VPAD // n_sub
        sid = lax.axis_index("s")
        pltpu.sync_copy(s_hbm.at[pl.ds(sid * sl, sl)],
                        s_sh.at[pl.ds(sid * sl, sl)])
        plsc.subcore_barrier()
        for k in range(_DEPTH):
            pltpu.async_copy(s_sh.at[idx_v.at[k]], vals_v.at[k], sem)

        @pl.loop(0, n_chunks - _DEPTH)
        def _pipe(j):
            pltpu.make_async_copy(s_sh.at[idx_v.at[j]], vals_v.at[j], sem).wait()
            jn = j + _DEPTH
            pltpu.async_copy(s_sh.at[idx_v.at[jn]], vals_v.at[jn], sem)

        for k in range(_DEPTH):
            pltpu.make_async_copy(s_sh.at[idx_v.at[k]], vals_v.at[k], sem).wait()

        iota16 = lax.iota(jnp.int32, 16)

        @pl.loop(0, rows_per_w // 16)
        def _rows(g):
            off = (g * 16 + iota16) * seq
            maxv = plsc.load_gather(vals_v, [off >> 7, off & 127])
            amax = jnp.zeros((16,), jnp.int32)
            for l in range(1, seq):
                o = off + l
                val = plsc.load_gather(vals_v, [o >> 7, o & 127])
                upd = val > maxv
                maxv = jnp.where(upd, val, maxv)
                amax = jnp.where(upd, jnp.full((16,), l, jnp.int32), amax)
            amax_v[pl.ds(g * 16, 16)] = amax

        pltpu.sync_copy(amax_v, out_hbm.at[pl.ds(wid * rows_per_w, rows_per_w)])

    return sc_argmax


def kernel(x, table, W, b):
    batch, seq = x.shape
    s_flat = _score_table(table.T, W.reshape(-1, 1))
    rows_per_w = batch // _NW
    n_chunks = (rows_per_w * seq) // _CHUNK
    x_r = x.astype(jnp.int32).reshape(_NW, n_chunks, _CHUNK)
    return _make_sc_argmax(batch, seq, rows_per_w)(x_r, s_flat)


# speedup vs baseline: 16.5100x; 1.1183x over previous
import functools

import jax
import jax.numpy as jnp
from jax import lax
from jax.experimental import pallas as pl
from jax.experimental.pallas import tpu as pltpu
import jax.experimental.pallas.tpu_sc as plsc

_NC, _NS = 2, 16
_NW = _NC * _NS
_CHUNK = 128
_DEPTH = 16
_LANE_CHUNK = 65536
_VPAD = 1 << 20


def _score_body(tab_ref, w_ref, out_ref):
    s = jnp.sum(tab_ref[...] * w_ref[...], axis=0, keepdims=True)
    vv = (pl.program_id(0) * _LANE_CHUNK
          + lax.broadcasted_iota(jnp.int32, (1, _LANE_CHUNK), 1))
    out_ref[...] = jnp.where(vv == 0, 0.0, s)


def _score_table(tab_t, w_col):
    d, v = tab_t.shape
    out = pl.pallas_call(
        _score_body,
        grid=(_VPAD // _LANE_CHUNK,),
        in_specs=[
            pl.BlockSpec((d, _LANE_CHUNK), lambda i: (0, i)),
            pl.BlockSpec((d, 1), lambda i: (0, 0)),
        ],
        out_specs=pl.BlockSpec((1, _LANE_CHUNK), lambda i: (0, i)),
        out_shape=jax.ShapeDtypeStruct((1, _VPAD), jnp.float32),
    )(tab_t, w_col)
    return out.reshape(_VPAD)


def _make_sc_argmax(batch, seq, rows_per_w):
    n_idx = rows_per_w * seq
    n_chunks = n_idx // _CHUNK
    assert n_chunks * _CHUNK == n_idx and rows_per_w % 16 == 0
    mesh = plsc.VectorSubcoreMesh(core_axis_name="c", subcore_axis_name="s")

    n_sub = _NS
    tpw = rows_per_w // _CHUNK

    @functools.partial(
        pl.kernel,
        out_type=jax.ShapeDtypeStruct((batch,), jnp.int32),
        mesh=mesh,
        compiler_params=pltpu.CompilerParams(needs_layout_passes=False),
        scratch_types=[
            pltpu.VMEM((seq, tpw, _CHUNK), jnp.int32),
            pltpu.VMEM((n_idx,), jnp.float32),
            pltpu.VMEM((rows_per_w,), jnp.int32),
            pltpu.VMEM_SHARED((_VPAD,), jnp.float32),
            pltpu.SemaphoreType.DMA,
        ],
    )
    def sc_argmax(xq_hbm, s_hbm, out_hbm, idx_v, vals_v, amax_v, s_sh, sem):
        wid = lax.axis_index("s") * _NC + lax.axis_index("c")
        pltpu.sync_copy(xq_hbm.at[:, pl.ds(wid * tpw, tpw), :], idx_v)
        sl = _VPAD // n_sub
        sid = lax.axis_index("s")
        pltpu.sync_copy(s_hbm.at[pl.ds(sid * sl, sl)],
                        s_sh.at[pl.ds(sid * sl, sl)])
        plsc.subcore_barrier()

        def chunk_idx(j):
            return idx_v.at[lax.div(j, tpw), lax.rem(j, tpw)]

        for k in range(_DEPTH):
            pltpu.async_copy(s_sh.at[chunk_idx(k)],
                             vals_v.at[pl.ds(k * _CHUNK, _CHUNK)], sem)

        @pl.loop(0, n_chunks - _DEPTH)
        def _pipe(j):
            pltpu.make_async_copy(s_sh.at[chunk_idx(j)],
                                  vals_v.at[pl.ds(j * _CHUNK, _CHUNK)], sem).wait()
            jn = j + _DEPTH
            pltpu.async_copy(s_sh.at[chunk_idx(jn)],
                             vals_v.at[pl.ds(jn * _CHUNK, _CHUNK)], sem)

        for k in range(_DEPTH):
            pltpu.make_async_copy(s_sh.at[chunk_idx(k)],
                                  vals_v.at[pl.ds(k * _CHUNK, _CHUNK)], sem).wait()

        @pl.loop(0, rows_per_w // 16)
        def _rows(g):
            b0 = g * 16
            maxv = vals_v[pl.ds(b0, 16)]
            amax = jnp.zeros((16,), jnp.int32)
            for l in range(1, seq):
                val = vals_v[pl.ds(l * rows_per_w + b0, 16)]
                upd = val > maxv
                maxv = jnp.where(upd, val, maxv)
                amax = jnp.where(upd, jnp.full((16,), l, jnp.int32), amax)
            amax_v[pl.ds(b0, 16)] = amax

        pltpu.sync_copy(amax_v, out_hbm.at[pl.ds(wid * rows_per_w, rows_per_w)])

    return sc_argmax


def kernel(x, table, W, b):
    batch, seq = x.shape
    s_flat = _score_table(table.T, W.reshape(-1, 1))
    rows_per_w = batch // _NW
    x_q = x.astype(jnp.int32).T.reshape(seq, batch // _CHUNK, _CHUNK)
    return _make_sc_argmax(batch, seq, rows_per_w)(x_q, s_flat)
